# baseline (speedup 1.0000x reference)
import jax
import jax.numpy as jnp
from jax.experimental import pallas as pl
from jax.experimental.pallas import tpu as pltpu

EPS = 1e-12


def _proj_ln(x_ref, w_ref, bias_ref, pos_ref, gamma_ref, beta_ref, out_ref):
    x16 = x_ref[...].astype(jnp.bfloat16)
    y = jnp.dot(x16, w_ref[...], preferred_element_type=jnp.float32)
    y = y + bias_ref[...] + pos_ref[...]
    mean = jnp.mean(y, axis=-1, keepdims=True)
    yc = y - mean
    var = jnp.mean(yc * yc, axis=-1, keepdims=True)
    inv = jax.lax.rsqrt(var + EPS)
    out_ref[0] = yc * inv * gamma_ref[...] + beta_ref[...]


def _proj_ln_acc(acc_ref, *rest):
    del acc_ref
    _proj_ln(*rest)


def _stream_call(acc, x, w, bias2, position_table, gamma2, beta2,
                 tb, tok_off, T):
    B, N, D = x.shape
    H = w.shape[1]
    n_tiles = N // tb
    off = tok_off // tb
    x_rows = x.reshape(B * N, D)

    data_specs = [
        pl.BlockSpec((tb, D), lambda t, b: (b * n_tiles + t, 0)),
        pl.BlockSpec((D, H), lambda t, b: (0, 0)),
        pl.BlockSpec((1, H), lambda t, b: (0, 0)),
        pl.BlockSpec((tb, H), lambda t, b: (t + off, 0)),
        pl.BlockSpec((1, H), lambda t, b: (0, 0)),
        pl.BlockSpec((1, H), lambda t, b: (0, 0)),
    ]
    out_spec = pl.BlockSpec((1, tb, H), lambda t, b: (b, t + off, 0))
    out_shape = jax.ShapeDtypeStruct((B, T, H), jnp.float32)

    if acc is None:
        return pl.pallas_call(
            _proj_ln,
            grid=(n_tiles, B),
            in_specs=data_specs,
            out_specs=out_spec,
            out_shape=out_shape,
        )(x_rows, w, bias2, position_table, gamma2, beta2)
    return pl.pallas_call(
        _proj_ln_acc,
        grid=(n_tiles, B),
        in_specs=[pl.BlockSpec(memory_space=pl.ANY)] + data_specs,
        out_specs=out_spec,
        out_shape=out_shape,
        input_output_aliases={0: 0},
    )(acc, x_rows, w, bias2, position_table, gamma2, beta2)


def kernel(obj, rel, frm, act, W_obj, b_obj, W_rel, b_rel, W_frame, b_frame,
           W_action, b_action, token_type_table, position_table, ln_gamma, ln_beta):
    B, NO, D = obj.shape
    NR, NF, NA = rel.shape[1], frm.shape[1], act.shape[1]
    T = NO + NR + NF + NA
    H = W_obj.shape[1]

    gamma2 = ln_gamma.reshape(1, H)
    beta2 = ln_beta.reshape(1, H)

    def pick_tb(n, off):
        for tb in (512, 256, 128, 64, 32):
            if n % tb == 0 and off % tb == 0:
                return tb
        raise ValueError(f"stream length {n} at offset {off} not tileable")

    streams = [
        (obj, W_obj, b_obj, 1, pick_tb(NO, 0), 0),
        (rel, W_rel, b_rel, 2, pick_tb(NR, NO), NO),
        (frm, W_frame, b_frame, 3, pick_tb(NF, NO + NR), NO + NR),
        (act, W_action, b_action, 4, pick_tb(NA, NO + NR + NF), NO + NR + NF),
    ]
    acc = None
    for x, w, b, tt_row, tb, tok_off in streams:
        w16 = w.astype(jnp.bfloat16)
        bias2 = (b + token_type_table[tt_row]).reshape(1, H)
        acc = _stream_call(acc, x, w16, bias2, position_table, gamma2, beta2,
                           tb, tok_off, T)

    non_pad_mask = jnp.ones((B, T), dtype=bool)
    return acc, non_pad_mask


# speedup vs baseline: 1.3044x; 1.3044x over previous
import jax
import jax.numpy as jnp
from jax.experimental import pallas as pl
from jax.experimental.pallas import tpu as pltpu

EPS = 1e-12


def _proj_ln(x_ref, w_ref, bias_ref, pos_ref, gamma_ref, beta_ref, out_ref):
    w = w_ref[...]
    add = bias_ref[...] + pos_ref[...]
    gamma = gamma_ref[...]
    beta = beta_ref[...]
    for i in range(x_ref.shape[0]):
        x16 = x_ref[i].astype(jnp.bfloat16)
        y = jnp.dot(x16, w, preferred_element_type=jnp.float32)
        y = y + add
        mean = jnp.mean(y, axis=-1, keepdims=True)
        yc = y - mean
        var = jnp.mean(yc * yc, axis=-1, keepdims=True)
        inv = jax.lax.rsqrt(var + EPS)
        out_ref[i] = yc * inv * gamma + beta


def _proj_ln_acc(acc_ref, *rest):
    del acc_ref
    _proj_ln(*rest)


def _stream_call(acc, x, w, bias2, position_table, gamma2, beta2,
                 tb, tok_off, T):
    B, N, D = x.shape
    H = w.shape[1]
    n_tiles = N // tb
    off = tok_off // tb

    data_specs = [
        pl.BlockSpec((B, tb, D), lambda t: (0, t, 0)),
        pl.BlockSpec((D, H), lambda t: (0, 0)),
        pl.BlockSpec((1, H), lambda t: (0, 0)),
        pl.BlockSpec((tb, H), lambda t: (t + off, 0)),
        pl.BlockSpec((1, H), lambda t: (0, 0)),
        pl.BlockSpec((1, H), lambda t: (0, 0)),
    ]
    out_spec = pl.BlockSpec((B, tb, H), lambda t: (0, t + off, 0))
    out_shape = jax.ShapeDtypeStruct((B, T, H), jnp.float32)

    if acc is None:
        return pl.pallas_call(
            _proj_ln,
            grid=(n_tiles,),
            in_specs=data_specs,
            out_specs=out_spec,
            out_shape=out_shape,
        )(x, w, bias2, position_table, gamma2, beta2)
    return pl.pallas_call(
        _proj_ln_acc,
        grid=(n_tiles,),
        in_specs=[pl.BlockSpec(memory_space=pl.ANY)] + data_specs,
        out_specs=out_spec,
        out_shape=out_shape,
        input_output_aliases={0: 0},
    )(acc, x, w, bias2, position_table, gamma2, beta2)


def kernel(obj, rel, frm, act, W_obj, b_obj, W_rel, b_rel, W_frame, b_frame,
           W_action, b_action, token_type_table, position_table, ln_gamma, ln_beta):
    B, NO, D = obj.shape
    NR, NF, NA = rel.shape[1], frm.shape[1], act.shape[1]
    T = NO + NR + NF + NA
    H = W_obj.shape[1]

    gamma2 = ln_gamma.reshape(1, H)
    beta2 = ln_beta.reshape(1, H)

    def pick_tb(n, off):
        for tb in (256, 128, 64, 32):
            if n % tb == 0 and off % tb == 0:
                return tb
        raise ValueError(f"stream length {n} at offset {off} not tileable")

    streams = [
        (obj, W_obj, b_obj, 1, pick_tb(NO, 0), 0),
        (rel, W_rel, b_rel, 2, pick_tb(NR, NO), NO),
        (frm, W_frame, b_frame, 3, pick_tb(NF, NO + NR), NO + NR),
        (act, W_action, b_action, 4, pick_tb(NA, NO + NR + NF), NO + NR + NF),
    ]
    acc = None
    for x, w, b, tt_row, tb, tok_off in streams:
        w16 = w.astype(jnp.bfloat16)
        bias2 = (b + token_type_table[tt_row]).reshape(1, H)
        acc = _stream_call(acc, x, w16, bias2, position_table, gamma2, beta2,
                           tb, tok_off, T)

    non_pad_mask = jnp.ones((B, T), dtype=bool)
    return acc, non_pad_mask


# speedup vs baseline: 1.3784x; 1.0567x over previous
import jax
import jax.numpy as jnp
from jax.experimental import pallas as pl
from jax.experimental.pallas import tpu as pltpu

EPS = 1e-12


def _proj_ln(x_ref, w_ref, bias_ref, pos_ref, gamma_ref, beta_ref, out_ref,
             w16_ref):
    @pl.when(pl.program_id(0) == 0)
    def _cast_weights():
        w16_ref[...] = w_ref[...].astype(jnp.bfloat16)

    w = w16_ref[...]
    add = bias_ref[...] + pos_ref[...]
    gamma = gamma_ref[...]
    beta = beta_ref[...]
    for i in range(x_ref.shape[0]):
        x16 = x_ref[i].astype(jnp.bfloat16)
        y = jnp.dot(x16, w, preferred_element_type=jnp.float32)
        y = y + add
        mean = jnp.mean(y, axis=-1, keepdims=True)
        yc = y - mean
        var = jnp.mean(yc * yc, axis=-1, keepdims=True)
        inv = jax.lax.rsqrt(var + EPS)
        out_ref[i] = yc * inv * gamma + beta


def _proj_ln_acc(acc_ref, *rest):
    del acc_ref
    _proj_ln(*rest)


def _stream_call(acc, x, w, bias2, position_table, gamma2, beta2,
                 tb, tok_off, T):
    B, N, D = x.shape
    H = w.shape[1]
    n_tiles = N // tb
    off = tok_off // tb

    data_specs = [
        pl.BlockSpec((B, tb, D), lambda t: (0, t, 0)),
        pl.BlockSpec((D, H), lambda t: (0, 0)),
        pl.BlockSpec((1, H), lambda t: (0, 0)),
        pl.BlockSpec((tb, H), lambda t: (t + off, 0)),
        pl.BlockSpec((1, H), lambda t: (0, 0)),
        pl.BlockSpec((1, H), lambda t: (0, 0)),
    ]
    out_spec = pl.BlockSpec((B, tb, H), lambda t: (0, t + off, 0))
    out_shape = jax.ShapeDtypeStruct((B, T, H), jnp.float32)

    scratch = [pltpu.VMEM(w.shape, jnp.bfloat16)]
    if acc is None:
        return pl.pallas_call(
            _proj_ln,
            grid=(n_tiles,),
            in_specs=data_specs,
            out_specs=out_spec,
            out_shape=out_shape,
            scratch_shapes=scratch,
        )(x, w, bias2, position_table, gamma2, beta2)
    return pl.pallas_call(
        _proj_ln_acc,
        grid=(n_tiles,),
        in_specs=[pl.BlockSpec(memory_space=pl.ANY)] + data_specs,
        out_specs=out_spec,
        out_shape=out_shape,
        input_output_aliases={0: 0},
        scratch_shapes=scratch,
    )(acc, x, w, bias2, position_table, gamma2, beta2)


def kernel(obj, rel, frm, act, W_obj, b_obj, W_rel, b_rel, W_frame, b_frame,
           W_action, b_action, token_type_table, position_table, ln_gamma, ln_beta):
    B, NO, D = obj.shape
    NR, NF, NA = rel.shape[1], frm.shape[1], act.shape[1]
    T = NO + NR + NF + NA
    H = W_obj.shape[1]

    gamma2 = ln_gamma.reshape(1, H)
    beta2 = ln_beta.reshape(1, H)

    def pick_tb(n, off):
        for tb in (256, 128, 64, 32):
            if n % tb == 0 and off % tb == 0:
                return tb
        raise ValueError(f"stream length {n} at offset {off} not tileable")

    streams = [
        (obj, W_obj, b_obj, 1, pick_tb(NO, 0), 0),
        (rel, W_rel, b_rel, 2, pick_tb(NR, NO), NO),
        (frm, W_frame, b_frame, 3, pick_tb(NF, NO + NR), NO + NR),
        (act, W_action, b_action, 4, pick_tb(NA, NO + NR + NF), NO + NR + NF),
    ]
    acc = None
    for x, w, b, tt_row, tb, tok_off in streams:
        bias2 = (b + token_type_table[tt_row]).reshape(1, H)
        acc = _stream_call(acc, x, w, bias2, position_table, gamma2, beta2,
                           tb, tok_off, T)

    non_pad_mask = jnp.ones((B, T), dtype=bool)
    return acc, non_pad_mask


# speedup vs baseline: 1.5209x; 1.1034x over previous
import jax
import jax.numpy as jnp
from jax.experimental import pallas as pl
from jax.experimental.pallas import tpu as pltpu

EPS = 1e-12
TB = 256


def _make_body(tiles, seg_first, B, D, H):
    n_steps = len(tiles)

    def body(obj_ref, rel_ref, frm_ref, act_ref,
             w0_ref, w1_ref, w2_ref, w3_ref,
             bias_ref, pos_ref, gamma_ref, beta_ref, out_ref,
             xbuf, obuf, posbuf, wf32, wcur, biascur,
             in_sem, pos_sem, out_sem, w_sem):
        t = pl.program_id(0)
        xsrcs = (obj_ref, rel_ref, frm_ref, act_ref)
        wsrcs = (w0_ref, w1_ref, w2_ref, w3_ref)

        def x_copy(k):
            s, lstart, valid, _ = tiles[k]
            p = k % 2
            return pltpu.make_async_copy(
                xsrcs[s].at[:, pl.ds(lstart, valid), :],
                xbuf.at[p, :, 0:valid, :], in_sem.at[p])

        def pos_copy(k):
            _, _, valid, gstart = tiles[k]
            p = k % 2
            return pltpu.make_async_copy(
                pos_ref.at[pl.ds(gstart, valid), :],
                posbuf.at[p, 0:valid, :], pos_sem.at[p])

        def out_copy(k):
            _, _, valid, gstart = tiles[k]
            p = k % 2
            return pltpu.make_async_copy(
                obuf.at[p, :, 0:valid, :],
                out_ref.at[:, pl.ds(gstart, valid), :], out_sem.at[p])

        def w_copy(s):
            return pltpu.make_async_copy(wsrcs[s], wf32.at[s], w_sem.at[s])

        @pl.when(t == 0)
        def _prologue():
            for s in range(len(wsrcs)):
                w_copy(s).start()
            x_copy(0).start()
            pos_copy(0).start()

        for k in range(1, n_steps):
            @pl.when(t == k - 1)
            def _prefetch(k=k):
                x_copy(k).start()
                pos_copy(k).start()

        for s, first in enumerate(seg_first):
            @pl.when(t == first)
            def _load_weights(s=s):
                w_copy(s).wait()
                wcur[...] = wf32[s].astype(jnp.bfloat16)
                biascur[...] = bias_ref[s]

        for k in range(n_steps):
            @pl.when(t == k)
            def _wait_in(k=k):
                x_copy(k).wait()
                pos_copy(k).wait()

        for k in range(n_steps - 2):
            @pl.when(t == k + 2)
            def _wait_out(k=k):
                out_copy(k).wait()

        def compute(p):
            add = biascur[...] + posbuf[p]
            gamma = gamma_ref[...]
            beta = beta_ref[...]
            w = wcur[...]
            for i in range(B):
                x16 = xbuf[p, i].astype(jnp.bfloat16)
                y = jnp.dot(x16, w, preferred_element_type=jnp.float32)
                y = y + add
                mean = jnp.mean(y, axis=-1, keepdims=True)
                yc = y - mean
                var = jnp.mean(yc * yc, axis=-1, keepdims=True)
                inv = jax.lax.rsqrt(var + EPS)
                obuf[p, i] = yc * inv * gamma + beta

        @pl.when(jax.lax.rem(t, 2) == 0)
        def _compute_even():
            compute(0)

        @pl.when(jax.lax.rem(t, 2) == 1)
        def _compute_odd():
            compute(1)

        for k in range(n_steps):
            @pl.when(t == k)
            def _ship(k=k):
                out_copy(k).start()

        @pl.when(t == n_steps - 1)
        def _epilogue():
            if n_steps >= 2:
                out_copy(n_steps - 2).wait()
            out_copy(n_steps - 1).wait()

    return body


def kernel(obj, rel, frm, act, W_obj, b_obj, W_rel, b_rel, W_frame, b_frame,
           W_action, b_action, token_type_table, position_table, ln_gamma, ln_beta):
    B, NO, D = obj.shape
    NR, NF, NA = rel.shape[1], frm.shape[1], act.shape[1]
    T = NO + NR + NF + NA
    H = W_obj.shape[1]

    tiles = []
    seg_first = []
    tok = 0
    for s, n in enumerate((NO, NR, NF, NA)):
        seg_first.append(len(tiles))
        for j in range(0, n, TB):
            valid = min(TB, n - j)
            tiles.append((s, j, valid, tok))
            tok += valid

    bias_all = jnp.stack([
        b_obj + token_type_table[1],
        b_rel + token_type_table[2],
        b_frame + token_type_table[3],
        b_action + token_type_table[4],
    ]).reshape(4, 1, H)
    gamma2 = ln_gamma.reshape(1, H)
    beta2 = ln_beta.reshape(1, H)

    any_spec = pl.BlockSpec(memory_space=pl.ANY)
    in_specs = (
        [any_spec] * 4
        + [any_spec] * 4
        + [pl.BlockSpec((4, 1, H), lambda t: (0, 0, 0)),
           any_spec,
           pl.BlockSpec((1, H), lambda t: (0, 0)),
           pl.BlockSpec((1, H), lambda t: (0, 0))]
    )
    scratch_shapes = [
        pltpu.VMEM((2, B, TB, D), jnp.float32),
        pltpu.VMEM((2, B, TB, H), jnp.float32),
        pltpu.VMEM((2, TB, H), jnp.float32),
        pltpu.VMEM((4, D, H), jnp.float32),
        pltpu.VMEM((D, H), jnp.bfloat16),
        pltpu.VMEM((1, H), jnp.float32),
        pltpu.SemaphoreType.DMA((2,)),
        pltpu.SemaphoreType.DMA((2,)),
        pltpu.SemaphoreType.DMA((2,)),
        pltpu.SemaphoreType.DMA((4,)),
    ]

    out = pl.pallas_call(
        _make_body(tiles, seg_first, B, D, H),
        grid=(len(tiles),),
        in_specs=in_specs,
        out_specs=pl.BlockSpec(memory_space=pl.ANY),
        out_shape=jax.ShapeDtypeStruct((B, T, H), jnp.float32),
        scratch_shapes=scratch_shapes,
    )(obj, rel, frm, act, W_obj, W_rel, W_frame, W_action,
      bias_all, position_table, gamma2, beta2)

    non_pad_mask = jnp.ones((B, T), dtype=bool)
    return out, non_pad_mask


# speedup vs baseline: 1.5223x; 1.0009x over previous
import jax
import jax.numpy as jnp
from jax.experimental import pallas as pl
from jax.experimental.pallas import tpu as pltpu

EPS = 1e-12
TB = 256


def _make_body(tiles, seg_first, B, D, H):
    n_steps = len(tiles)

    def body(obj_ref, rel_ref, frm_ref, act_ref,
             w0_ref, w1_ref, w2_ref, w3_ref,
             bias_ref, pos_ref, gamma_ref, beta_ref, out_ref,
             xbuf, obuf, posbuf, wf32, wcur, biascur,
             in_sem, pos_sem, out_sem, w_sem):
        t = pl.program_id(0)
        xsrcs = (obj_ref, rel_ref, frm_ref, act_ref)
        wsrcs = (w0_ref, w1_ref, w2_ref, w3_ref)

        def x_copy(k):
            s, lstart, valid, _ = tiles[k]
            p = k % 2
            hb = B // 2
            return [pltpu.make_async_copy(
                xsrcs[s].at[pl.ds(c * hb, hb), pl.ds(lstart, valid), :],
                xbuf.at[p, pl.ds(c * hb, hb), 0:valid, :], in_sem.at[p])
                for c in range(2)]

        def pos_copy(k):
            _, _, valid, gstart = tiles[k]
            p = k % 2
            return pltpu.make_async_copy(
                pos_ref.at[pl.ds(gstart, valid), :],
                posbuf.at[p, 0:valid, :], pos_sem.at[p])

        def out_copy(k):
            _, _, valid, gstart = tiles[k]
            p = k % 2
            hb = B // 2
            return [pltpu.make_async_copy(
                obuf.at[p, pl.ds(c * hb, hb), 0:valid, :],
                out_ref.at[pl.ds(c * hb, hb), pl.ds(gstart, valid), :],
                out_sem.at[p])
                for c in range(2)]

        def w_copy(s):
            return pltpu.make_async_copy(wsrcs[s], wf32.at[s], w_sem.at[s])

        @pl.when(t == 0)
        def _prologue():
            for s in range(len(wsrcs)):
                w_copy(s).start()
            for c in x_copy(0):
                c.start()
            pos_copy(0).start()

        for k in range(1, n_steps):
            @pl.when(t == k - 1)
            def _prefetch(k=k):
                for c in x_copy(k):
                    c.start()
                pos_copy(k).start()

        for s, first in enumerate(seg_first):
            @pl.when(t == first)
            def _load_weights(s=s):
                w_copy(s).wait()
                wcur[...] = wf32[s].astype(jnp.bfloat16)
                biascur[...] = bias_ref[s]

        for k in range(n_steps):
            @pl.when(t == k)
            def _wait_in(k=k):
                for c in x_copy(k):
                    c.wait()
                pos_copy(k).wait()

        for k in range(n_steps - 2):
            @pl.when(t == k + 2)
            def _wait_out(k=k):
                for c in out_copy(k):
                    c.wait()

        def compute(p):
            add = biascur[...] + posbuf[p]
            gamma = gamma_ref[...]
            beta = beta_ref[...]
            w = wcur[...]
            for i in range(B):
                x16 = xbuf[p, i].astype(jnp.bfloat16)
                y = jnp.dot(x16, w, preferred_element_type=jnp.float32)
                y = y + add
                mean = jnp.mean(y, axis=-1, keepdims=True)
                yc = y - mean
                var = jnp.mean(yc * yc, axis=-1, keepdims=True)
                inv = jax.lax.rsqrt(var + EPS)
                obuf[p, i] = yc * inv * gamma + beta

        @pl.when(jax.lax.rem(t, 2) == 0)
        def _compute_even():
            compute(0)

        @pl.when(jax.lax.rem(t, 2) == 1)
        def _compute_odd():
            compute(1)

        for k in range(n_steps):
            @pl.when(t == k)
            def _ship(k=k):
                for c in out_copy(k):
                    c.start()

        @pl.when(t == n_steps - 1)
        def _epilogue():
            if n_steps >= 2:
                for c in out_copy(n_steps - 2):
                    c.wait()
            for c in out_copy(n_steps - 1):
                c.wait()

    return body


def kernel(obj, rel, frm, act, W_obj, b_obj, W_rel, b_rel, W_frame, b_frame,
           W_action, b_action, token_type_table, position_table, ln_gamma, ln_beta):
    B, NO, D = obj.shape
    NR, NF, NA = rel.shape[1], frm.shape[1], act.shape[1]
    T = NO + NR + NF + NA
    H = W_obj.shape[1]

    tiles = []
    seg_first = []
    tok = 0
    for s, n in enumerate((NO, NR, NF, NA)):
        seg_first.append(len(tiles))
        for j in range(0, n, TB):
            valid = min(TB, n - j)
            tiles.append((s, j, valid, tok))
            tok += valid

    bias_all = jnp.stack([
        b_obj + token_type_table[1],
        b_rel + token_type_table[2],
        b_frame + token_type_table[3],
        b_action + token_type_table[4],
    ]).reshape(4, 1, H)
    gamma2 = ln_gamma.reshape(1, H)
    beta2 = ln_beta.reshape(1, H)

    any_spec = pl.BlockSpec(memory_space=pl.ANY)
    in_specs = (
        [any_spec] * 4
        + [any_spec] * 4
        + [pl.BlockSpec((4, 1, H), lambda t: (0, 0, 0)),
           any_spec,
           pl.BlockSpec((1, H), lambda t: (0, 0)),
           pl.BlockSpec((1, H), lambda t: (0, 0))]
    )
    scratch_shapes = [
        pltpu.VMEM((2, B, TB, D), jnp.float32),
        pltpu.VMEM((2, B, TB, H), jnp.float32),
        pltpu.VMEM((2, TB, H), jnp.float32),
        pltpu.VMEM((4, D, H), jnp.float32),
        pltpu.VMEM((D, H), jnp.bfloat16),
        pltpu.VMEM((1, H), jnp.float32),
        pltpu.SemaphoreType.DMA((2,)),
        pltpu.SemaphoreType.DMA((2,)),
        pltpu.SemaphoreType.DMA((2,)),
        pltpu.SemaphoreType.DMA((4,)),
    ]

    out = pl.pallas_call(
        _make_body(tiles, seg_first, B, D, H),
        grid=(len(tiles),),
        in_specs=in_specs,
        out_specs=pl.BlockSpec(memory_space=pl.ANY),
        out_shape=jax.ShapeDtypeStruct((B, T, H), jnp.float32),
        scratch_shapes=scratch_shapes,
    )(obj, rel, frm, act, W_obj, W_rel, W_frame, W_action,
      bias_all, position_table, gamma2, beta2)

    non_pad_mask = jnp.ones((B, T), dtype=bool)
    return out, non_pad_mask
